# baseline (speedup 1.0000x reference)
import jax
import jax.numpy as jnp
from jax.experimental import pallas as pl

_BM = 512


def _mv_block(x_ref, s_ref, ind_ref, joint_ref):
    ind_ref[...] = jax.lax.dot_general(
        x_ref[...], s_ref[...], (((1,), (0,)), ((), ())),
        preferred_element_type=jnp.float32)
    joint_ref[...] = jnp.zeros_like(joint_ref)


def kernel(xs_0, xs_1, xs_2, xs_3, layer_tab, type_tab, mod_tab, score,
           W1, b1, W2, b2):
    B, NP = xs_3.shape
    individual, joint = pl.pallas_call(
        _mv_block,
        grid=(B // _BM,),
        in_specs=[
            pl.BlockSpec((_BM, NP), lambda i: (i, 0)),
            pl.BlockSpec((NP, 1), lambda i: (0, 0)),
        ],
        out_specs=[
            pl.BlockSpec((_BM, 1), lambda i: (i, 0)),
            pl.BlockSpec((_BM, 1), lambda i: (i, 0)),
        ],
        out_shape=[
            jax.ShapeDtypeStruct((B, 1), jnp.float32),
            jax.ShapeDtypeStruct((B, 1), jnp.float32),
        ],
    )(xs_3, score)
    return (individual, joint)


# speedup vs baseline: 1.0026x; 1.0026x over previous
import jax
import jax.numpy as jnp
from jax.experimental import pallas as pl

_BM = 512
_NSPLIT = 4


def _mv_block(*refs):
    x_refs = refs[:_NSPLIT]
    s_ref, ind_ref, joint_ref = refs[_NSPLIT:]
    sub = _BM // _NSPLIT
    for k, xr in enumerate(x_refs):
        ind_ref[k * sub:(k + 1) * sub, :] = jax.lax.dot_general(
            xr[...], s_ref[...], (((1,), (0,)), ((), ())),
            preferred_element_type=jnp.float32)
    joint_ref[...] = jnp.zeros_like(joint_ref)


def kernel(xs_0, xs_1, xs_2, xs_3, layer_tab, type_tab, mod_tab, score,
           W1, b1, W2, b2):
    B, NP = xs_3.shape
    sub = _BM // _NSPLIT
    x_specs = [
        pl.BlockSpec((sub, NP), lambda i, k=k: (_NSPLIT * i + k, 0))
        for k in range(_NSPLIT)
    ]
    individual, joint = pl.pallas_call(
        _mv_block,
        grid=(B // _BM,),
        in_specs=x_specs + [pl.BlockSpec((NP, 1), lambda i: (0, 0))],
        out_specs=[
            pl.BlockSpec((_BM, 1), lambda i: (i, 0)),
            pl.BlockSpec((_BM, 1), lambda i: (i, 0)),
        ],
        out_shape=[
            jax.ShapeDtypeStruct((B, 1), jnp.float32),
            jax.ShapeDtypeStruct((B, 1), jnp.float32),
        ],
    )(*([xs_3] * _NSPLIT), score)
    return (individual, joint)


# speedup vs baseline: 1.0111x; 1.0084x over previous
import jax
import jax.numpy as jnp
from jax.experimental import pallas as pl
from jax.experimental.pallas import tpu as pltpu

_BM = 512
_NSPLIT = 4


def _mv_block(*refs):
    x_refs = refs[:_NSPLIT]
    s_ref, ind_ref, joint_ref = refs[_NSPLIT:]
    sub = _BM // _NSPLIT
    for k, xr in enumerate(x_refs):
        ind_ref[k * sub:(k + 1) * sub, :] = jax.lax.dot_general(
            xr[...], s_ref[...], (((1,), (0,)), ((), ())),
            preferred_element_type=jnp.float32)
    joint_ref[...] = jnp.zeros_like(joint_ref)


def kernel(xs_0, xs_1, xs_2, xs_3, layer_tab, type_tab, mod_tab, score,
           W1, b1, W2, b2):
    B, NP = xs_3.shape
    sub = _BM // _NSPLIT
    x_specs = [
        pl.BlockSpec((sub, NP), lambda i, k=k: (_NSPLIT * i + k, 0))
        for k in range(_NSPLIT)
    ]
    individual, joint = pl.pallas_call(
        _mv_block,
        grid=(B // _BM,),
        in_specs=x_specs + [pl.BlockSpec((NP, 1), lambda i: (0, 0))],
        out_specs=[
            pl.BlockSpec((_BM, 1), lambda i: (i, 0)),
            pl.BlockSpec((_BM, 1), lambda i: (i, 0)),
        ],
        out_shape=[
            jax.ShapeDtypeStruct((B, 1), jnp.float32),
            jax.ShapeDtypeStruct((B, 1), jnp.float32),
        ],
        compiler_params=pltpu.CompilerParams(
            dimension_semantics=("parallel",)),
    )(*([xs_3] * _NSPLIT), score)
    return (individual, joint)


# speedup vs baseline: 3.1606x; 3.1259x over previous
import jax
import jax.numpy as jnp
from jax.experimental import pallas as pl
from jax.experimental.pallas import tpu as pltpu

_NK = 200


def _mv_cols(x_ref, s_ref, ind_ref, joint_ref):
    k = pl.program_id(0)
    part = jnp.sum(x_ref[...] * s_ref[...], axis=0, keepdims=True)

    @pl.when(k == 0)
    def _init():
        ind_ref[...] = part
        joint_ref[...] = jnp.zeros_like(joint_ref)

    @pl.when(k > 0)
    def _acc():
        ind_ref[...] += part


def kernel(xs_0, xs_1, xs_2, xs_3, layer_tab, type_tab, mod_tab, score,
           W1, b1, W2, b2):
    B, NP = xs_3.shape
    xt = xs_3.T
    individual, joint = pl.pallas_call(
        _mv_cols,
        grid=(NP // _NK,),
        in_specs=[
            pl.BlockSpec((_NK, B), lambda k: (k, 0)),
            pl.BlockSpec((_NK, 1), lambda k: (k, 0)),
        ],
        out_specs=[
            pl.BlockSpec((1, B), lambda k: (0, 0)),
            pl.BlockSpec((1, B), lambda k: (0, 0)),
        ],
        out_shape=[
            jax.ShapeDtypeStruct((1, B), jnp.float32),
            jax.ShapeDtypeStruct((1, B), jnp.float32),
        ],
        compiler_params=pltpu.CompilerParams(
            dimension_semantics=("arbitrary",)),
    )(xt, score)
    return (individual.reshape(B, 1), joint.reshape(B, 1))
